# baseline (speedup 1.0000x reference)
import functools

import jax
import jax.numpy as jnp
from jax import lax
from jax.experimental import pallas as pl
from jax.experimental.pallas import tpu as pltpu
from jax.experimental.pallas import tpu_sc as plsc

BATCH = 16384
EMBED_DIM = 32
N_ROWS = 1000000
NUM_CORES = 2
NUM_SUBCORES = 16
NUM_WORKERS = NUM_CORES * NUM_SUBCORES
BPW = BATCH // NUM_WORKERS
LANES = 16
TCT = (N_ROWS + 127) // 128
TCPW = 245
NCHUNK = 31
CHCOLS = 8 * 128
MAXOFF = 999040
CAP = 64
MCAP = 800
STAGE_ROWS = BATCH + 8


def _phase1_body(uidx_hbm, iidx_hbm, uemb_hbm, iemb_hbm,
                 ustage_hbm, istage_hbm,
                 idxb_v, jl_v, rl_v, bj_v, br_v, jscr_v,
                 chunkb_v, valb_v, cnt_s, semc):
    wid = lax.axis_index("s") * NUM_CORES + lax.axis_index("c")
    lo_tc = wid * TCPW
    hi_tc = jnp.minimum(lo_tc + TCPW, TCT)
    iota = lax.iota(jnp.int32, LANES)

    for table in range(2):
        idx_hbm = (uidx_hbm, iidx_hbm)[table]
        emb_hbm = (uemb_hbm, iemb_hbm)[table]
        stage_hbm = (ustage_hbm, istage_hbm)[table]

        pltpu.sync_copy(idx_hbm, idxb_v)

        def chunk_off(c):
            return jnp.minimum((lo_tc + c * 8) * 128, MAXOFF)

        pltpu.async_copy(
            emb_hbm.at[pl.ds(0, 32),
                       pl.ds(pl.multiple_of(chunk_off(0), 128), CHCOLS)],
            chunkb_v.at[0], semc.at[0])

        def scang(g, cur):
            rvec = idxb_v[pl.ds(g * LANES, LANES)]
            tc = rvec >> 7
            m = (tc >= lo_tc) & (tc < hi_tc)
            jv = g * LANES + iota
            plsc.store_compressed(jl_v.at[pl.ds(cur, LANES)], jv, mask=m)
            plsc.store_compressed(rl_v.at[pl.ds(cur, LANES)], rvec, mask=m)
            cnt = plsc.all_reduce_population_count(m)
            return jnp.minimum(cur + cnt[0], MCAP - LANES)

        n_m = lax.fori_loop(0, BATCH // LANES, scang, jnp.int32(0))

        dummy = lax.broadcast(jnp.int32(BATCH), (LANES,))

        def initb(b, carry):
            for q in range(CAP // LANES):
                bj_v[b, pl.ds(q * LANES, LANES)] = dummy
            return carry

        lax.fori_loop(0, NCHUNK, initb, 0)
        for b in range(NCHUNK):
            cnt_s[b] = 0

        def buckg(e, carry):
            jv = jl_v[pl.ds(e * LANES, LANES)]
            rv = rl_v[pl.ds(e * LANES, LANES)]
            valid = (e * LANES + iota) < n_m
            ch = ((rv >> 7) - lo_tc) >> 3
            for b in range(NCHUNK):
                mb = valid & (ch == b)
                cnt = plsc.all_reduce_population_count(mb)
                cur = cnt_s[b]
                plsc.store_compressed(bj_v.at[b, pl.ds(cur, LANES)], jv, mask=mb)
                plsc.store_compressed(br_v.at[b, pl.ds(cur, LANES)], rv, mask=mb)
                cnt_s[b] = jnp.minimum(cur + cnt[0], CAP - LANES)
            return carry

        lax.fori_loop(0, (n_m + LANES - 1) // LANES, buckg, 0)

        def chunk(c, carry):
            p = c % 2

            @pl.when(c < NCHUNK - 1)
            def _():
                off = pl.multiple_of(chunk_off(c + 1), 128)
                pltpu.async_copy(
                    emb_hbm.at[pl.ds(0, 32), pl.ds(off, CHCOLS)],
                    chunkb_v.at[(c + 1) % 2], semc.at[(c + 1) % 2])

            pltpu.make_async_copy(
                emb_hbm.at[pl.ds(0, 32), pl.ds(0, CHCOLS)],
                chunkb_v.at[p], semc.at[p]).wait()

            off_c = chunk_off(c)
            pv = lax.broadcast(p, (LANES,))
            ngr = (cnt_s[c] + LANES - 1) >> 4

            def egroup(grp, carry2):
                jvec = bj_v[c, pl.ds(grp * LANES, LANES)]
                rvec = br_v[c, pl.ds(grp * LANES, LANES)]
                jscr_v[pl.ds(0, LANES)] = jvec
                for k in range(LANES):
                    col = jnp.clip(rvec[k] - off_c, 0, CHCOLS - 1)
                    colv = lax.broadcast(col, (LANES,))
                    lov = plsc.load_gather(chunkb_v, [pv, iota, colv])
                    hiv = plsc.load_gather(chunkb_v, [pv, iota + LANES, colv])
                    valb_v[k, pl.ds(0, LANES)] = lov
                    valb_v[k, pl.ds(LANES, LANES)] = hiv
                pltpu.sync_copy(valb_v, stage_hbm.at[jscr_v])
                return carry2

            lax.fori_loop(0, ngr, egroup, 0)
            return carry

        lax.fori_loop(0, NCHUNK, chunk, 0)


def _phase2_body(ustage_hbm, istage_hbm, out_hbm, ub_v, ib_v, outb_v):
    wid = lax.axis_index("s") * NUM_CORES + lax.axis_index("c")
    iota = lax.iota(jnp.int32, LANES)
    tr_lo = iota >> 3
    s_lo = iota & 7
    tr_hi = (iota + LANES) >> 3

    for h in range(2):
        base = wid * BPW + h * 256
        pltpu.sync_copy(ustage_hbm.at[pl.ds(base, 256)], ub_v)
        pltpu.sync_copy(istage_hbm.at[pl.ds(base, 256)], ib_v)

        def g2(g, carry):
            for k in range(LANES):
                jloc = g * LANES + k
                u_lo = ub_v[jloc, pl.ds(0, LANES)]
                u_hi = ub_v[jloc, pl.ds(LANES, LANES)]
                v_lo = ib_v[jloc, pl.ds(0, LANES)]
                v_hi = ib_v[jloc, pl.ds(LANES, LANES)]
                j = h * 256 + jloc
                tcb = lax.broadcast(j >> 7, (LANES,))
                lj = lax.broadcast(j & 127, (LANES,))
                plsc.store_scatter(outb_v, [tr_lo, tcb, s_lo, lj],
                                   u_lo * v_lo)
                plsc.store_scatter(outb_v, [tr_hi, tcb, s_lo, lj],
                                   u_hi * v_hi)
            return carry

        lax.fori_loop(0, 256 // LANES, g2, 0)

    for tr in range(4):
        pltpu.sync_copy(outb_v.at[tr], out_hbm.at[tr, pl.ds(wid * 4, 4)])


@jax.jit
def _gmf(uidx, iidx, uemb, iemb):
    mesh = plsc.VectorSubcoreMesh(core_axis_name="c", subcore_axis_name="s")
    stage_t = jax.ShapeDtypeStruct((STAGE_ROWS, 128), jnp.float32)
    p1 = functools.partial(
        pl.kernel,
        mesh=mesh,
        out_type=(stage_t, stage_t),
        scratch_types=[
            pltpu.VMEM((BATCH,), jnp.int32),
            pltpu.VMEM((MCAP,), jnp.int32),
            pltpu.VMEM((MCAP,), jnp.int32),
            pltpu.VMEM((NCHUNK, CAP), jnp.int32),
            pltpu.VMEM((NCHUNK, CAP), jnp.int32),
            pltpu.VMEM((LANES,), jnp.int32),
            pltpu.VMEM((2, 32, CHCOLS), jnp.float32),
            pltpu.VMEM((LANES, 128), jnp.float32),
            pltpu.SMEM((NCHUNK + 1,), jnp.int32),
            pltpu.SemaphoreType.DMA((2,)),
        ],
        compiler_params=pltpu.CompilerParams(
            use_tc_tiling_on_sc=True, needs_layout_passes=False),
    )(_phase1_body)
    ustage, istage = p1(uidx, iidx, uemb, iemb)

    p2 = functools.partial(
        pl.kernel,
        mesh=mesh,
        out_type=jax.ShapeDtypeStruct((4, 128, 8, 128), jnp.float32),
        scratch_types=[
            pltpu.VMEM((256, 128), jnp.float32),
            pltpu.VMEM((256, 128), jnp.float32),
            pltpu.VMEM((4, 4, 8, 128), jnp.float32),
        ],
        compiler_params=pltpu.CompilerParams(
            use_tc_tiling_on_sc=True, needs_layout_passes=False),
    )(_phase2_body)
    return p2(ustage, istage)


def kernel(user_idx, item_idx, user_emb, item_emb):
    u3 = user_emb.T
    v3 = item_emb.T
    out4 = _gmf(user_idx.astype(jnp.int32), item_idx.astype(jnp.int32),
                u3, v3)
    return out4.transpose(1, 3, 0, 2).reshape(BATCH, EMBED_DIM)


# speedup vs baseline: 4.2001x; 4.2001x over previous
import functools

import jax
import jax.numpy as jnp
from jax import lax
from jax.experimental import pallas as pl
from jax.experimental.pallas import tpu as pltpu
from jax.experimental.pallas import tpu_sc as plsc

BATCH = 16384
EMBED_DIM = 32
N_ROWS = 1000000
NUM_CORES = 2
NUM_SUBCORES = 16
NUM_WORKERS = NUM_CORES * NUM_SUBCORES
BPW = BATCH // NUM_WORKERS
LANES = 16
NGROUPS = BPW // LANES
NBUF = 8


def _gmf_body(uidx_hbm, iidx_hbm, uemb_hbm, iemb_hbm, out_hbm,
              uraw_v, iraw_v, ubuf_v, ibuf_v, outb_v, sem):
    wid = lax.axis_index("s") * NUM_CORES + lax.axis_index("c")
    base = wid * BPW

    pltpu.sync_copy(uidx_hbm.at[pl.ds(base, BPW)], uraw_v)
    pltpu.sync_copy(iidx_hbm.at[pl.ds(base, BPW)], iraw_v)

    iota = lax.iota(jnp.int32, LANES)
    tr_lo = iota >> 3
    s_lo = iota & 7
    tr_hi = (iota + LANES) >> 3

    def fire(ru, ri, slot):
        off_u = pl.multiple_of((ru >> 7) * 128, 128)
        pltpu.async_copy(
            uemb_hbm.at[pl.ds(0, 32), pl.ds(off_u, 128)],
            ubuf_v.at[slot], sem.at[slot])
        off_i = pl.multiple_of((ri >> 7) * 128, 128)
        pltpu.async_copy(
            iemb_hbm.at[pl.ds(0, 32), pl.ds(off_i, 128)],
            ibuf_v.at[slot], sem.at[slot])

    def drain(slot):
        pltpu.make_async_copy(
            uemb_hbm.at[pl.ds(0, 32), pl.ds(0, 128)],
            ubuf_v.at[slot], sem.at[slot]).wait()
        pltpu.make_async_copy(
            iemb_hbm.at[pl.ds(0, 32), pl.ds(0, 128)],
            ibuf_v.at[slot], sem.at[slot]).wait()

    head_u = uraw_v[pl.ds(0, LANES)]
    head_i = iraw_v[pl.ds(0, LANES)]
    for k in range(NBUF):
        fire(head_u[k], head_i[k], k)

    def group(g, carry):
        cur_u = uraw_v[pl.ds(g * LANES, LANES)]
        cur_i = iraw_v[pl.ds(g * LANES, LANES)]
        nxt = jnp.minimum((g + 1) * LANES, BPW - LANES)
        nxt_u = uraw_v[pl.ds(nxt, LANES)]
        nxt_i = iraw_v[pl.ds(nxt, LANES)]
        for k in range(LANES):
            j = g * LANES + k
            slot = k % NBUF
            drain(slot)
            ru = cur_u[k]
            ri = cur_i[k]
            lu = lax.broadcast(ru & 127, (LANES,))
            li = lax.broadcast(ri & 127, (LANES,))
            slotv = lax.broadcast(jnp.int32(slot), (LANES,))
            u_lo = plsc.load_gather(ubuf_v, [slotv, iota, lu])
            u_hi = plsc.load_gather(ubuf_v, [slotv, iota + LANES, lu])
            v_lo = plsc.load_gather(ibuf_v, [slotv, iota, li])
            v_hi = plsc.load_gather(ibuf_v, [slotv, iota + LANES, li])
            tcb = lax.broadcast(j >> 7, (LANES,))
            lj = lax.broadcast(j & 127, (LANES,))
            plsc.store_scatter(outb_v, [tr_lo, tcb, s_lo, lj], u_lo * v_lo)
            plsc.store_scatter(outb_v, [tr_hi, tcb, s_lo, lj], u_hi * v_hi)
            if k < LANES - NBUF:
                fire(cur_u[k + NBUF], cur_i[k + NBUF], slot)
            else:
                kk = k + NBUF - LANES

                @pl.when(g < NGROUPS - 1)
                def _():
                    fire(nxt_u[kk], nxt_i[kk], slot)
        return carry

    lax.fori_loop(0, NGROUPS, group, 0)

    for tr in range(4):
        pltpu.sync_copy(outb_v.at[tr], out_hbm.at[tr, pl.ds(wid * 4, 4)])


@jax.jit
def _gmf(uidx, iidx, uemb, iemb):
    mesh = plsc.VectorSubcoreMesh(core_axis_name="c", subcore_axis_name="s")
    run = functools.partial(
        pl.kernel,
        mesh=mesh,
        out_type=jax.ShapeDtypeStruct((4, 128, 8, 128), jnp.float32),
        scratch_types=[
            pltpu.VMEM((BPW,), jnp.int32),
            pltpu.VMEM((BPW,), jnp.int32),
            pltpu.VMEM((NBUF, 32, 128), jnp.float32),
            pltpu.VMEM((NBUF, 32, 128), jnp.float32),
            pltpu.VMEM((4, 4, 8, 128), jnp.float32),
            pltpu.SemaphoreType.DMA((NBUF,)),
        ],
        compiler_params=pltpu.CompilerParams(
            use_tc_tiling_on_sc=True, needs_layout_passes=False),
    )(_gmf_body)
    return run(uidx, iidx, uemb, iemb)


def kernel(user_idx, item_idx, user_emb, item_emb):
    u3 = user_emb.T
    v3 = item_emb.T
    out4 = _gmf(user_idx.astype(jnp.int32), item_idx.astype(jnp.int32),
                u3, v3)
    return out4.transpose(1, 3, 0, 2).reshape(BATCH, EMBED_DIM)
